# baseline (speedup 1.0000x reference)
import functools

import jax
import jax.numpy as jnp
from jax import lax
from jax.experimental import pallas as pl
from jax.experimental.pallas import tpu as pltpu
from jax.experimental.pallas import tpu_sc as plsc

BATCH = 16384
NENT = 1000000
EDIM = 64
RDIM = 32
NREL = 1000
WROW = EDIM * RDIM + 128
NC = 2
NS = 16
NW = NC * NS
ROWS_PER_TILE = BATCH // NW
BLOCK = 128
N_BLOCKS = ROWS_PER_TILE // BLOCK
SUB = 16
N_SUB = BLOCK // SUB

_PIB = jax.lax.GatherScatterMode.PROMISE_IN_BOUNDS
_GATHER_DNUMS = lax.GatherDimensionNumbers(
    offset_dims=(), collapsed_slice_dims=(0,), start_index_map=(0,))


def _bcast(vec, lane):
    idx = jnp.broadcast_to(jnp.asarray(lane, jnp.int32), (16,))[:, None]
    return lax.gather(vec, idx, _GATHER_DNUMS, slice_sizes=(1,), mode=_PIB)


def _sc_body(h_hbm, r_hbm, p_hbm, n_hbm, ent_hbm, wre_hbm,
             diff_hbm, reg_hbm,
             h_idx, r_idx, p_idx, n_idx,
             h_buf, p_buf, n_buf, w_buf, score_buf, reg_stage, sem):
    wid = lax.axis_index("s") * NC + lax.axis_index("c")
    base = wid * ROWS_PER_TILE
    zeros = jnp.zeros((16,), jnp.float32)

    def block_body(b, carry):
        racc0, racc1 = carry
        row0 = base + b * BLOCK
        pltpu.sync_copy(h_hbm.at[pl.ds(row0, BLOCK)], h_idx)
        pltpu.sync_copy(r_hbm.at[pl.ds(row0, BLOCK)], r_idx)
        pltpu.sync_copy(p_hbm.at[pl.ds(row0, BLOCK)], p_idx)
        pltpu.sync_copy(n_hbm.at[pl.ds(row0, BLOCK)], n_idx)
        g1 = pltpu.async_copy(ent_hbm.at[h_idx], h_buf, sem)
        g2 = pltpu.async_copy(ent_hbm.at[p_idx], p_buf, sem)
        g3 = pltpu.async_copy(ent_hbm.at[n_idx], n_buf, sem)
        g1.wait(); g2.wait(); g3.wait()

        for s in range(N_SUB):
            gw = pltpu.async_copy(wre_hbm.at[r_idx.at[pl.ds(s * SUB, SUB)]],
                                  w_buf, sem)
            gw.wait()

            def row_body(i, rc, s=s):
                racc0, racc1 = rc
                j = s * SUB + i
                uq = []
                vq = []
                for q in range(4):
                    off = q * 16
                    hq = h_buf[j, pl.ds(off, 16)]
                    pq = p_buf[j, pl.ds(off, 16)]
                    nq = n_buf[j, pl.ds(off, 16)]
                    uq.append(hq - pq)
                    vq.append(hq - nq)
                    racc0 = racc0 + hq * hq + pq * pq + nq * nq
                re0 = w_buf[i, pl.ds(2048, 16)]
                re1 = w_buf[i, pl.ds(2064, 16)]
                racc1 = racc1 + re0 * re0 + re1 * re1
                ap0 = re0
                ap1 = re1
                an0 = re0
                an1 = re1
                for d in range(EDIM):
                    w0 = w_buf[i, pl.ds(d * RDIM, 16)]
                    w1 = w_buf[i, pl.ds(d * RDIM + 16, 16)]
                    ub = _bcast(uq[d // 16], d % 16)
                    vb = _bcast(vq[d // 16], d % 16)
                    ap0 = ap0 + ub * w0
                    ap1 = ap1 + ub * w1
                    an0 = an0 + vb * w0
                    an1 = an1 + vb * w1
                spn = ap0 * ap0 + ap1 * ap1 - an0 * an0 - an1 * an1
                score_buf[j, :] = spn
                return (racc0, racc1)

            racc0, racc1 = lax.fori_loop(0, SUB, row_body, (racc0, racc1))

        pltpu.sync_copy(score_buf, diff_hbm.at[pl.ds(row0, BLOCK)])
        return (racc0, racc1)

    racc0, racc1 = lax.fori_loop(0, N_BLOCKS, block_body, (zeros, zeros))
    reg_stage[pl.ds(0, 16)] = racc0
    reg_stage[pl.ds(16, 16)] = racc1
    for k in range(2, 8):
        reg_stage[pl.ds(k * 16, 16)] = zeros
    pltpu.sync_copy(reg_stage, reg_hbm.at[wid])


def _tc_body(diff_ref, reg_ref, out_ref):
    spn = diff_ref[...]
    z = 0.5 * jnp.sum(spn, axis=1, keepdims=True)
    nz = -z
    softplus = jnp.maximum(nz, 0.0) + jnp.log1p(jnp.exp(-jnp.abs(nz)))
    kg = jnp.sum(softplus) * (1.0 / BATCH)
    regt = jnp.sum(reg_ref[...]) * (1.0 / (2.0 * BATCH))
    out_ref[0, 0] = kg + 0.01 * regt


def kernel(h, r, pos_t, neg_t, entity_embed, relation_embed, relation_weight):
    ent128 = jnp.concatenate([entity_embed, entity_embed], axis=1)
    wre = jnp.concatenate(
        [relation_weight.reshape(NREL, EDIM * RDIM),
         jnp.pad(relation_embed, ((0, 0), (0, 128 - RDIM)))], axis=1)
    mesh = plsc.VectorSubcoreMesh(core_axis_name="c", subcore_axis_name="s")
    sc = pl.kernel(
        _sc_body,
        mesh=mesh,
        out_type=(
            jax.ShapeDtypeStruct((BATCH, 16), jnp.float32),
            jax.ShapeDtypeStruct((NW, 128), jnp.float32),
        ),
        scratch_types=[
            pltpu.VMEM((BLOCK,), jnp.int32),
            pltpu.VMEM((BLOCK,), jnp.int32),
            pltpu.VMEM((BLOCK,), jnp.int32),
            pltpu.VMEM((BLOCK,), jnp.int32),
            pltpu.VMEM((BLOCK, 128), jnp.float32),
            pltpu.VMEM((BLOCK, 128), jnp.float32),
            pltpu.VMEM((BLOCK, 128), jnp.float32),
            pltpu.VMEM((SUB, WROW), jnp.float32),
            pltpu.VMEM((BLOCK, 16), jnp.float32),
            pltpu.VMEM((128,), jnp.float32),
            pltpu.SemaphoreType.DMA,
        ],
    )
    diff, reg = sc(h, r, pos_t, neg_t, ent128, wre)
    out = pl.pallas_call(
        _tc_body,
        out_shape=jax.ShapeDtypeStruct((1, 1), jnp.float32),
        out_specs=pl.BlockSpec(memory_space=pltpu.SMEM),
    )(diff, reg)
    return out[0, 0]


# speedup vs baseline: 1.6193x; 1.6193x over previous
import functools

import jax
import jax.numpy as jnp
from jax import lax
from jax.experimental import pallas as pl
from jax.experimental.pallas import tpu as pltpu
from jax.experimental.pallas import tpu_sc as plsc

BATCH = 16384
NENT = 1000000
EDIM = 64
RDIM = 32
NREL = 1000
WROW = EDIM * RDIM + 128
NC = 2
NS = 16
NW = NC * NS
ROWS_PER_TILE = BATCH // NW
BLOCK = 128
N_BLOCKS = ROWS_PER_TILE // BLOCK
SUB = 8
N_SUB = BLOCK // SUB

_PIB = jax.lax.GatherScatterMode.PROMISE_IN_BOUNDS
_GATHER_DNUMS = lax.GatherDimensionNumbers(
    offset_dims=(), collapsed_slice_dims=(0,), start_index_map=(0,))


def _bcast(vec, lane):
    idx = jnp.broadcast_to(jnp.asarray(lane, jnp.int32), (16,))[:, None]
    return lax.gather(vec, idx, _GATHER_DNUMS, slice_sizes=(1,), mode=_PIB)


def _sc_body(h_hbm, r_hbm, p_hbm, n_hbm, ent_hbm, wre_hbm,
             diff_hbm, reg_hbm,
             h_idx, r_idx, p_idx, n_idx,
             h_buf, p_buf, n_buf, w_buf, score_buf, reg_stage,
             sem, sem_w0, sem_w1):
    wid = lax.axis_index("s") * NC + lax.axis_index("c")
    base = wid * ROWS_PER_TILE
    zeros = jnp.zeros((16,), jnp.float32)

    def make_row_body(wslot):
        def row_body(i, rc):
            racc0, racc1, s = rc
            j = s * SUB + i
            uq = []
            vq = []
            for q in range(4):
                off = q * 16
                hq = h_buf[j, pl.ds(off, 16)]
                pq = p_buf[j, pl.ds(off, 16)]
                nq = n_buf[j, pl.ds(off, 16)]
                uq.append(hq - pq)
                vq.append(hq - nq)
                racc0 = racc0 + hq * hq + pq * pq + nq * nq
            re0 = w_buf[wslot, i, pl.ds(2048, 16)]
            re1 = w_buf[wslot, i, pl.ds(2064, 16)]
            racc1 = racc1 + re0 * re0 + re1 * re1
            ap0 = re0
            ap1 = re1
            an0 = re0
            an1 = re1
            for d in range(EDIM):
                w0 = w_buf[wslot, i, pl.ds(d * RDIM, 16)]
                w1 = w_buf[wslot, i, pl.ds(d * RDIM + 16, 16)]
                ub = _bcast(uq[d // 16], d % 16)
                vb = _bcast(vq[d // 16], d % 16)
                ap0 = ap0 + ub * w0
                ap1 = ap1 + ub * w1
                an0 = an0 + vb * w0
                an1 = an1 + vb * w1
            spn = ap0 * ap0 + ap1 * ap1 - an0 * an0 - an1 * an1
            score_buf[j, :] = spn
            return (racc0, racc1, s)
        return row_body

    _row0 = make_row_body(0)
    _row1 = make_row_body(1)

    def fire_w(s, wslot, wsem):
        pltpu.async_copy(wre_hbm.at[r_idx.at[pl.ds(s * SUB, SUB)]],
                         w_buf.at[wslot], wsem)

    def drain_w(wslot, wsem):
        pltpu.make_async_copy(wre_hbm.at[pl.ds(0, SUB)],
                              w_buf.at[wslot], wsem).wait()

    def block_body(b, carry):
        racc0, racc1 = carry
        row0 = base + b * BLOCK
        c1 = pltpu.async_copy(h_hbm.at[pl.ds(row0, BLOCK)], h_idx, sem)
        c2 = pltpu.async_copy(r_hbm.at[pl.ds(row0, BLOCK)], r_idx, sem)
        c3 = pltpu.async_copy(p_hbm.at[pl.ds(row0, BLOCK)], p_idx, sem)
        c4 = pltpu.async_copy(n_hbm.at[pl.ds(row0, BLOCK)], n_idx, sem)
        c1.wait(); c2.wait(); c3.wait(); c4.wait()
        g1 = pltpu.async_copy(ent_hbm.at[h_idx], h_buf, sem)
        g2 = pltpu.async_copy(ent_hbm.at[p_idx], p_buf, sem)
        g3 = pltpu.async_copy(ent_hbm.at[n_idx], n_buf, sem)
        fire_w(0, 0, sem_w0)
        g1.wait(); g2.wait(); g3.wait()

        def pair_body(k, rc):
            racc0, racc1 = rc
            s_even = 2 * k
            fire_w(s_even + 1, 1, sem_w1)
            drain_w(0, sem_w0)
            racc0, racc1, _ = lax.fori_loop(0, SUB, _row0,
                                            (racc0, racc1, s_even))
            @pl.when(k < N_SUB // 2 - 1)
            def _():
                fire_w(s_even + 2, 0, sem_w0)
            drain_w(1, sem_w1)
            racc0, racc1, _ = lax.fori_loop(0, SUB, _row1,
                                            (racc0, racc1, s_even + 1))
            return (racc0, racc1)

        racc0, racc1 = lax.fori_loop(0, N_SUB // 2, pair_body,
                                     (racc0, racc1))
        pltpu.sync_copy(score_buf, diff_hbm.at[pl.ds(row0, BLOCK)])
        return (racc0, racc1)

    racc0, racc1 = lax.fori_loop(0, N_BLOCKS, block_body, (zeros, zeros))
    reg_stage[pl.ds(0, 16)] = racc0
    reg_stage[pl.ds(16, 16)] = racc1
    for k in range(2, 8):
        reg_stage[pl.ds(k * 16, 16)] = zeros
    pltpu.sync_copy(reg_stage, reg_hbm.at[wid])


def _tc_body(diff_ref, reg_ref, out_ref):
    spn = diff_ref[...]
    z = 0.5 * jnp.sum(spn, axis=1, keepdims=True)
    nz = -z
    softplus = jnp.maximum(nz, 0.0) + jnp.log1p(jnp.exp(-jnp.abs(nz)))
    kg = jnp.sum(softplus) * (1.0 / BATCH)
    regt = jnp.sum(reg_ref[...]) * (1.0 / (2.0 * BATCH))
    out_ref[0, 0] = kg + 0.01 * regt


def kernel(h, r, pos_t, neg_t, entity_embed, relation_embed, relation_weight):
    ent128 = jnp.dot(entity_embed, jnp.eye(EDIM, 128, dtype=jnp.float32),
                     precision=jax.lax.Precision.HIGHEST)
    wre = jnp.concatenate(
        [relation_weight.reshape(NREL, EDIM * RDIM),
         jnp.pad(relation_embed, ((0, 0), (0, 128 - RDIM)))], axis=1)
    mesh = plsc.VectorSubcoreMesh(core_axis_name="c", subcore_axis_name="s")
    sc = pl.kernel(
        _sc_body,
        mesh=mesh,
        out_type=(
            jax.ShapeDtypeStruct((BATCH, 16), jnp.float32),
            jax.ShapeDtypeStruct((NW, 128), jnp.float32),
        ),
        scratch_types=[
            pltpu.VMEM((BLOCK,), jnp.int32),
            pltpu.VMEM((BLOCK,), jnp.int32),
            pltpu.VMEM((BLOCK,), jnp.int32),
            pltpu.VMEM((BLOCK,), jnp.int32),
            pltpu.VMEM((BLOCK, 128), jnp.float32),
            pltpu.VMEM((BLOCK, 128), jnp.float32),
            pltpu.VMEM((BLOCK, 128), jnp.float32),
            pltpu.VMEM((2, SUB, WROW), jnp.float32),
            pltpu.VMEM((BLOCK, 16), jnp.float32),
            pltpu.VMEM((128,), jnp.float32),
            pltpu.SemaphoreType.DMA,
            pltpu.SemaphoreType.DMA,
            pltpu.SemaphoreType.DMA,
        ],
    )
    diff, reg = sc(h, r, pos_t, neg_t, ent128, wre)
    out = pl.pallas_call(
        _tc_body,
        out_shape=jax.ShapeDtypeStruct((1, 1), jnp.float32),
        out_specs=pl.BlockSpec(memory_space=pltpu.SMEM),
    )(diff, reg)
    return out[0, 0]


# speedup vs baseline: 2.3445x; 1.4478x over previous
import functools

import jax
import jax.numpy as jnp
from jax import lax
from jax.experimental import pallas as pl
from jax.experimental.pallas import tpu as pltpu
from jax.experimental.pallas import tpu_sc as plsc

BATCH = 16384
NENT = 1000000
EDIM = 64
RDIM = 32
NREL = 1000
WROW = EDIM * RDIM + 128
NC = 2
NS = 16
NW = NC * NS
ROWS_PER_TILE = BATCH // NW
BLOCK = 128
N_BLOCKS = ROWS_PER_TILE // BLOCK
SUB = 8
N_SUB = BLOCK // SUB

_PIB = jax.lax.GatherScatterMode.PROMISE_IN_BOUNDS
_GATHER_DNUMS = lax.GatherDimensionNumbers(
    offset_dims=(), collapsed_slice_dims=(0,), start_index_map=(0,))


def _bcast(vec, lane):
    idx = jnp.broadcast_to(jnp.asarray(lane, jnp.int32), (16,))[:, None]
    return lax.gather(vec, idx, _GATHER_DNUMS, slice_sizes=(1,), mode=_PIB)


def _sc_body(h_hbm, r_hbm, p_hbm, n_hbm, ent_hbm, wre_hbm,
             diff_hbm, reg_hbm,
             h_idx, r_idx, p_idx, n_idx,
             h_buf, p_buf, n_buf, w_buf, score_buf, reg_stage,
             sem, sem_w0, sem_w1):
    wid = lax.axis_index("s") * NC + lax.axis_index("c")
    base = wid * ROWS_PER_TILE
    zeros = jnp.zeros((16,), jnp.float32)

    def make_row_body(wslot):
        def row_body(i, rc):
            racc0, racc1, s = rc
            j = s * SUB + i
            uq = []
            vq = []
            for q in range(4):
                off = q * 16
                hq = h_buf[j, pl.ds(off, 16)]
                pq = p_buf[j, pl.ds(off, 16)]
                nq = n_buf[j, pl.ds(off, 16)]
                uq.append(hq - pq)
                vq.append(hq - nq)
                racc0 = racc0 + hq * hq + pq * pq + nq * nq
            re0 = w_buf[wslot, i, pl.ds(2048, 16)]
            re1 = w_buf[wslot, i, pl.ds(2064, 16)]
            racc1 = racc1 + re0 * re0 + re1 * re1
            ap0 = re0
            ap1 = re1
            an0 = re0
            an1 = re1
            for d in range(EDIM):
                w0 = w_buf[wslot, i, pl.ds(d * RDIM, 16)]
                w1 = w_buf[wslot, i, pl.ds(d * RDIM + 16, 16)]
                ub = _bcast(uq[d // 16], d % 16)
                vb = _bcast(vq[d // 16], d % 16)
                ap0 = ap0 + ub * w0
                ap1 = ap1 + ub * w1
                an0 = an0 + vb * w0
                an1 = an1 + vb * w1
            spn = ap0 * ap0 + ap1 * ap1 - an0 * an0 - an1 * an1
            score_buf[j, :] = spn
            return (racc0, racc1, s)
        return row_body

    _row0 = make_row_body(0)
    _row1 = make_row_body(1)

    def fire_w(s, wslot, wsem):
        pltpu.async_copy(wre_hbm.at[r_idx.at[pl.ds(s * SUB, SUB)]],
                         w_buf.at[wslot], wsem)

    def drain_w(wslot, wsem):
        pltpu.make_async_copy(wre_hbm.at[pl.ds(0, SUB)],
                              w_buf.at[wslot], wsem).wait()

    def block_body(b, carry):
        racc0, racc1 = carry
        row0 = base + b * BLOCK
        c1 = pltpu.async_copy(h_hbm.at[pl.ds(row0, BLOCK)], h_idx, sem)
        c2 = pltpu.async_copy(r_hbm.at[pl.ds(row0, BLOCK)], r_idx, sem)
        c3 = pltpu.async_copy(p_hbm.at[pl.ds(row0, BLOCK)], p_idx, sem)
        c4 = pltpu.async_copy(n_hbm.at[pl.ds(row0, BLOCK)], n_idx, sem)
        c1.wait(); c2.wait(); c3.wait(); c4.wait()
        g1 = pltpu.async_copy(ent_hbm.at[h_idx], h_buf, sem)
        g2 = pltpu.async_copy(ent_hbm.at[p_idx], p_buf, sem)
        g3 = pltpu.async_copy(ent_hbm.at[n_idx], n_buf, sem)
        fire_w(0, 0, sem_w0)
        g1.wait(); g2.wait(); g3.wait()

        def pair_body(k, rc):
            racc0, racc1 = rc
            s_even = 2 * k
            fire_w(s_even + 1, 1, sem_w1)
            drain_w(0, sem_w0)
            racc0, racc1, _ = lax.fori_loop(0, SUB, _row0,
                                            (racc0, racc1, s_even))
            @pl.when(k < N_SUB // 2 - 1)
            def _():
                fire_w(s_even + 2, 0, sem_w0)
            drain_w(1, sem_w1)
            racc0, racc1, _ = lax.fori_loop(0, SUB, _row1,
                                            (racc0, racc1, s_even + 1))
            return (racc0, racc1)

        racc0, racc1 = lax.fori_loop(0, N_SUB // 2, pair_body,
                                     (racc0, racc1))
        pltpu.sync_copy(score_buf, diff_hbm.at[pl.ds(row0, BLOCK)])
        return (racc0, racc1)

    racc0, racc1 = lax.fori_loop(0, N_BLOCKS, block_body, (zeros, zeros))
    reg_stage[pl.ds(0, 16)] = racc0
    reg_stage[pl.ds(16, 16)] = racc1
    for k in range(2, 8):
        reg_stage[pl.ds(k * 16, 16)] = zeros
    pltpu.sync_copy(reg_stage, reg_hbm.at[wid])


def _tc_body(diff_ref, reg_ref, out_ref):
    spn = diff_ref[...]
    z = 0.5 * jnp.sum(spn, axis=1, keepdims=True)
    nz = -z
    softplus = jnp.maximum(nz, 0.0) + jnp.log1p(jnp.exp(-jnp.abs(nz)))
    kg = jnp.sum(softplus) * (1.0 / BATCH)
    regt = jnp.sum(reg_ref[...]) * (1.0 / (2.0 * BATCH))
    out_ref[0, 0] = kg + 0.01 * regt


def kernel(h, r, pos_t, neg_t, entity_embed, relation_embed, relation_weight):
    ent128 = jnp.dot(entity_embed, jnp.eye(EDIM, 128, dtype=jnp.float32),
                     precision=jax.lax.Precision.DEFAULT)
    wre = jnp.concatenate(
        [relation_weight.reshape(NREL, EDIM * RDIM),
         jnp.pad(relation_embed, ((0, 0), (0, 128 - RDIM)))], axis=1)
    mesh = plsc.VectorSubcoreMesh(core_axis_name="c", subcore_axis_name="s")
    sc = pl.kernel(
        _sc_body,
        mesh=mesh,
        out_type=(
            jax.ShapeDtypeStruct((BATCH, 16), jnp.float32),
            jax.ShapeDtypeStruct((NW, 128), jnp.float32),
        ),
        scratch_types=[
            pltpu.VMEM((BLOCK,), jnp.int32),
            pltpu.VMEM((BLOCK,), jnp.int32),
            pltpu.VMEM((BLOCK,), jnp.int32),
            pltpu.VMEM((BLOCK,), jnp.int32),
            pltpu.VMEM((BLOCK, 128), jnp.float32),
            pltpu.VMEM((BLOCK, 128), jnp.float32),
            pltpu.VMEM((BLOCK, 128), jnp.float32),
            pltpu.VMEM((2, SUB, WROW), jnp.float32),
            pltpu.VMEM((BLOCK, 16), jnp.float32),
            pltpu.VMEM((128,), jnp.float32),
            pltpu.SemaphoreType.DMA,
            pltpu.SemaphoreType.DMA,
            pltpu.SemaphoreType.DMA,
        ],
    )
    diff, reg = sc(h, r, pos_t, neg_t, ent128, wre)
    out = pl.pallas_call(
        _tc_body,
        out_shape=jax.ShapeDtypeStruct((1, 1), jnp.float32),
        out_specs=pl.BlockSpec(memory_space=pltpu.SMEM),
    )(diff, reg)
    return out[0, 0]


# speedup vs baseline: 2.3482x; 1.0016x over previous
import jax
import jax.numpy as jnp
from jax import lax
from jax.experimental import pallas as pl
from jax.experimental.pallas import tpu as pltpu
from jax.experimental.pallas import tpu_sc as plsc

BATCH = 16384
NENT = 1000000
EDIM = 64
RDIM = 32
NREL = 1000
WROW = EDIM * RDIM + 128
NC = 2
NS = 16
NW = NC * NS
ROWS_PER_TILE = BATCH // NW
BLOCK = 128
N_BLOCKS = ROWS_PER_TILE // BLOCK
SUB = 8
N_SUB = BLOCK // SUB

_PIB = jax.lax.GatherScatterMode.PROMISE_IN_BOUNDS
_GATHER_DNUMS = lax.GatherDimensionNumbers(
    offset_dims=(), collapsed_slice_dims=(0,), start_index_map=(0,))


def _bcast(vec, lane):
    idx = jnp.broadcast_to(jnp.asarray(lane, jnp.int32), (16,))[:, None]
    return lax.gather(vec, idx, _GATHER_DNUMS, slice_sizes=(1,), mode=_PIB)


def _sc_body(h_hbm, r_hbm, p_hbm, n_hbm, ent_hbm, wre_hbm,
             diff_hbm, reg_hbm,
             h_idx, r_idx, p_idx, n_idx,
             h_buf, p_buf, n_buf, w_buf, score_buf, reg_stage,
             sem, sem_w0, sem_w1):
    wid = lax.axis_index("s") * NC + lax.axis_index("c")
    base = wid * ROWS_PER_TILE
    zeros = jnp.zeros((16,), jnp.float32)

    def make_row_body(wslot):
        def row_body(i, rc):
            racc0, racc1, s = rc
            j = s * SUB + i
            uq = []
            vq = []
            for q in range(4):
                off = q * 16
                hq = h_buf[j, pl.ds(off, 16)]
                pq = p_buf[j, pl.ds(off, 16)]
                nq = n_buf[j, pl.ds(off, 16)]
                uq.append(hq - pq)
                vq.append(hq - nq)
                racc0 = racc0 + hq * hq + pq * pq + nq * nq
            re0 = w_buf[wslot, i, pl.ds(2048, 16)]
            re1 = w_buf[wslot, i, pl.ds(2064, 16)]
            racc1 = racc1 + re0 * re0 + re1 * re1
            ap0 = re0
            ap1 = re1
            an0 = re0
            an1 = re1
            for d in range(EDIM):
                w0 = w_buf[wslot, i, pl.ds(d * RDIM, 16)]
                w1 = w_buf[wslot, i, pl.ds(d * RDIM + 16, 16)]
                ub = _bcast(uq[d // 16], d % 16)
                vb = _bcast(vq[d // 16], d % 16)
                ap0 = ap0 + ub * w0
                ap1 = ap1 + ub * w1
                an0 = an0 + vb * w0
                an1 = an1 + vb * w1
            spn = ap0 * ap0 + ap1 * ap1 - an0 * an0 - an1 * an1
            score_buf[j, :] = spn
            return (racc0, racc1, s)
        return row_body

    _row0 = make_row_body(0)
    _row1 = make_row_body(1)

    def fire_w(s, wslot, wsem):
        pltpu.async_copy(wre_hbm.at[r_idx.at[pl.ds(s * SUB, SUB)]],
                         w_buf.at[wslot], wsem)

    def drain_w(wslot, wsem):
        pltpu.make_async_copy(wre_hbm.at[pl.ds(0, SUB)],
                              w_buf.at[wslot], wsem).wait()

    def block_body(b, carry):
        racc0, racc1 = carry
        row0 = base + b * BLOCK
        c1 = pltpu.async_copy(h_hbm.at[pl.ds(row0, BLOCK)], h_idx, sem)
        c2 = pltpu.async_copy(r_hbm.at[pl.ds(row0, BLOCK)], r_idx, sem)
        c3 = pltpu.async_copy(p_hbm.at[pl.ds(row0, BLOCK)], p_idx, sem)
        c4 = pltpu.async_copy(n_hbm.at[pl.ds(row0, BLOCK)], n_idx, sem)
        c1.wait(); c2.wait(); c3.wait(); c4.wait()
        g1 = pltpu.async_copy(ent_hbm.at[h_idx], h_buf, sem)
        g2 = pltpu.async_copy(ent_hbm.at[p_idx], p_buf, sem)
        g3 = pltpu.async_copy(ent_hbm.at[n_idx], n_buf, sem)
        fire_w(0, 0, sem_w0)
        g1.wait(); g2.wait(); g3.wait()

        def pair_body(k, rc):
            racc0, racc1 = rc
            s_even = 2 * k
            fire_w(s_even + 1, 1, sem_w1)
            drain_w(0, sem_w0)
            racc0, racc1, _ = lax.fori_loop(0, SUB, _row0,
                                            (racc0, racc1, s_even))
            @pl.when(k < N_SUB // 2 - 1)
            def _():
                fire_w(s_even + 2, 0, sem_w0)
            drain_w(1, sem_w1)
            racc0, racc1, _ = lax.fori_loop(0, SUB, _row1,
                                            (racc0, racc1, s_even + 1))
            return (racc0, racc1)

        racc0, racc1 = lax.fori_loop(0, N_SUB // 2, pair_body,
                                     (racc0, racc1))
        pltpu.sync_copy(score_buf, diff_hbm.at[pl.ds(row0, BLOCK)])
        return (racc0, racc1)

    racc0, racc1 = lax.fori_loop(0, N_BLOCKS, block_body, (zeros, zeros))
    reg_stage[pl.ds(0, 16)] = racc0
    reg_stage[pl.ds(16, 16)] = racc1
    for k in range(2, 8):
        reg_stage[pl.ds(k * 16, 16)] = zeros
    pltpu.sync_copy(reg_stage, reg_hbm.at[wid])


def _tc_body(diff_ref, reg_ref, out_ref):
    spn = diff_ref[...]
    z = 0.5 * jnp.sum(spn, axis=1, keepdims=True)
    nz = -z
    softplus = jnp.maximum(nz, 0.0) + jnp.log1p(jnp.exp(-jnp.abs(nz)))
    kg = jnp.sum(softplus) * (1.0 / BATCH)
    regt = jnp.sum(reg_ref[...]) * (1.0 / (2.0 * BATCH))
    out_ref[0, 0] = kg + 0.01 * regt


def kernel(h, r, pos_t, neg_t, entity_embed, relation_embed, relation_weight):
    ent128 = jnp.dot(entity_embed, jnp.eye(EDIM, 128, dtype=jnp.float32),
                     precision=jax.lax.Precision.DEFAULT)
    wre = jnp.concatenate(
        [relation_weight.reshape(NREL, EDIM * RDIM),
         jnp.pad(relation_embed, ((0, 0), (0, 128 - RDIM)))], axis=1)
    mesh = plsc.VectorSubcoreMesh(core_axis_name="c", subcore_axis_name="s")
    sc = pl.kernel(
        _sc_body,
        mesh=mesh,
        out_type=(
            jax.ShapeDtypeStruct((BATCH, 16), jnp.float32),
            jax.ShapeDtypeStruct((NW, 128), jnp.float32),
        ),
        scratch_types=[
            pltpu.VMEM((BLOCK,), jnp.int32),
            pltpu.VMEM((BLOCK,), jnp.int32),
            pltpu.VMEM((BLOCK,), jnp.int32),
            pltpu.VMEM((BLOCK,), jnp.int32),
            pltpu.VMEM((BLOCK, 128), jnp.float32),
            pltpu.VMEM((BLOCK, 128), jnp.float32),
            pltpu.VMEM((BLOCK, 128), jnp.float32),
            pltpu.VMEM((2, SUB, WROW), jnp.float32),
            pltpu.VMEM((BLOCK, 16), jnp.float32),
            pltpu.VMEM((128,), jnp.float32),
            pltpu.SemaphoreType.DMA,
            pltpu.SemaphoreType.DMA,
            pltpu.SemaphoreType.DMA,
        ],
    )
    diff, reg = sc(h, r, pos_t, neg_t, ent128, wre)
    out = pl.pallas_call(
        _tc_body,
        out_shape=jax.ShapeDtypeStruct((1, 1), jnp.float32),
        out_specs=pl.BlockSpec(memory_space=pltpu.SMEM),
    )(diff, reg)
    return out[0, 0]
